# baseline (speedup 1.0000x reference)
import functools

import jax
import jax.numpy as jnp
from jax.experimental import pallas as pl
from jax.experimental.pallas import tpu as pltpu


def _round_up(x: int, m: int) -> int:
    return (x + m - 1) // m * m


def _gather_block_kernel(tok_ref, emb_hbm, out_ref, sem, *,
                         scale, block_tokens, chunk, unroll):
    base = pl.program_id(0) * block_tokens

    @pl.loop(0, block_tokens // unroll)
    def _(g):
        t0 = g * unroll
        toks = [tok_ref[base + t0 + u] for u in range(unroll)]
        for u in range(unroll):
            pltpu.make_async_copy(emb_hbm.at[toks[u]], out_ref.at[t0 + u],
                                  sem).start(priority=u & 1)

    pltpu.make_async_copy(emb_hbm.at[pl.ds(0, block_tokens)],
                          out_ref.at[...], sem).wait()
    out_ref[...] = out_ref[...] * scale


def _embed_gather(flat_tokens, emb_table, *, block_tokens, chunk, scale,
                  unroll):
    n_pad = flat_tokens.shape[0]
    V, D = emb_table.shape
    n_chunks = block_tokens // chunk
    return pl.pallas_call(
        functools.partial(_gather_block_kernel, scale=scale,
                          block_tokens=block_tokens, chunk=chunk,
                          unroll=unroll),
        out_shape=jax.ShapeDtypeStruct((n_pad, D), emb_table.dtype),
        grid_spec=pltpu.PrefetchScalarGridSpec(
            num_scalar_prefetch=1,
            grid=(n_pad // block_tokens,),
            in_specs=[pl.BlockSpec(memory_space=pl.ANY)],
            out_specs=pl.BlockSpec((block_tokens, D), lambda i, tok: (i, 0)),
            scratch_shapes=[pltpu.SemaphoreType.DMA],
        ),
        compiler_params=pltpu.CompilerParams(
            dimension_semantics=("parallel",),
            vmem_limit_bytes=48 << 20,
            disable_bounds_checks=True,
        ),
    )(flat_tokens, emb_table)


def kernel(tokens, emb_table):
    B, S = tokens.shape
    V, D = emb_table.shape
    N = B * S
    scale = float(D) ** 0.5

    block_tokens = 4096
    while block_tokens > N and block_tokens > 8:
        block_tokens //= 2
    chunk = min(256, block_tokens)
    unroll = 16 if block_tokens % 16 == 0 else 1

    n_pad = _round_up(N, block_tokens)
    flat = tokens.reshape(N).astype(jnp.int32)
    if n_pad != N:
        flat = jnp.concatenate([flat, jnp.zeros((n_pad - N,), jnp.int32)])

    out_flat = _embed_gather(flat, emb_table, block_tokens=block_tokens,
                             chunk=chunk, scale=scale, unroll=unroll)
    return out_flat[:N].reshape(B, S, D)


# speedup vs baseline: 1.0156x; 1.0156x over previous
import functools

import jax
import jax.numpy as jnp
from jax.experimental import pallas as pl
from jax.experimental.pallas import tpu as pltpu


def _round_up(x: int, m: int) -> int:
    return (x + m - 1) // m * m


def _gather_block_kernel(tok_ref, emb_hbm, out_ref, sem, *,
                         scale, block_tokens, chunk, unroll):
    base = pl.program_id(0) * block_tokens

    @pl.loop(0, block_tokens // unroll)
    def _(g):
        t0 = g * unroll
        toks = [tok_ref[base + t0 + u] for u in range(unroll)]
        for u in range(unroll):
            pltpu.make_async_copy(emb_hbm.at[toks[u]], out_ref.at[t0 + u],
                                  sem).start()

    pltpu.make_async_copy(emb_hbm.at[pl.ds(0, block_tokens)],
                          out_ref.at[...], sem).wait()
    out_ref[...] = out_ref[...] * scale


def _embed_gather(flat_tokens, emb_table, *, block_tokens, chunk, scale,
                  unroll):
    n_pad = flat_tokens.shape[0]
    V, D = emb_table.shape
    n_chunks = block_tokens // chunk
    return pl.pallas_call(
        functools.partial(_gather_block_kernel, scale=scale,
                          block_tokens=block_tokens, chunk=chunk,
                          unroll=unroll),
        out_shape=jax.ShapeDtypeStruct((n_pad, D), emb_table.dtype),
        grid_spec=pltpu.PrefetchScalarGridSpec(
            num_scalar_prefetch=1,
            grid=(n_pad // block_tokens,),
            in_specs=[pl.BlockSpec(memory_space=pl.ANY)],
            out_specs=pl.BlockSpec((block_tokens, D), lambda i, tok: (i, 0)),
            scratch_shapes=[pltpu.SemaphoreType.DMA],
        ),
        compiler_params=pltpu.CompilerParams(
            dimension_semantics=("parallel",),
            vmem_limit_bytes=48 << 20,
            disable_bounds_checks=True,
        ),
    )(flat_tokens, emb_table)


def kernel(tokens, emb_table):
    B, S = tokens.shape
    V, D = emb_table.shape
    N = B * S
    scale = float(D) ** 0.5

    block_tokens = 8192
    while block_tokens > N and block_tokens > 8:
        block_tokens //= 2
    chunk = min(256, block_tokens)
    unroll = 32 if block_tokens % 32 == 0 else 1

    n_pad = _round_up(N, block_tokens)
    flat = tokens.reshape(N).astype(jnp.int32)
    if n_pad != N:
        flat = jnp.concatenate([flat, jnp.zeros((n_pad - N,), jnp.int32)])

    out_flat = _embed_gather(flat, emb_table, block_tokens=block_tokens,
                             chunk=chunk, scale=scale, unroll=unroll)
    return out_flat[:N].reshape(B, S, D)


# speedup vs baseline: 1.0319x; 1.0161x over previous
import functools

import jax
import jax.numpy as jnp
from jax.experimental import pallas as pl
from jax.experimental.pallas import tpu as pltpu


def _round_up(x: int, m: int) -> int:
    return (x + m - 1) // m * m


def _gather_block_kernel(tok_ref, emb_hbm, out_ref, sem, *,
                         scale, block_tokens, chunk, unroll):
    base = pl.program_id(0) * block_tokens

    @pl.loop(0, block_tokens // unroll)
    def _(g):
        t0 = g * unroll
        toks = [tok_ref[base + t0 + u] for u in range(unroll)]
        for u in range(unroll):
            pltpu.make_async_copy(emb_hbm.at[toks[u]], out_ref.at[t0 + u],
                                  sem).start()

    pltpu.make_async_copy(emb_hbm.at[pl.ds(0, block_tokens)],
                          out_ref.at[...], sem).wait()
    out_ref[...] = out_ref[...] * scale


def _embed_gather(flat_tokens, emb_table, *, block_tokens, chunk, scale,
                  unroll):
    n_pad = flat_tokens.shape[0]
    V, D = emb_table.shape
    n_chunks = block_tokens // chunk
    return pl.pallas_call(
        functools.partial(_gather_block_kernel, scale=scale,
                          block_tokens=block_tokens, chunk=chunk,
                          unroll=unroll),
        out_shape=jax.ShapeDtypeStruct((n_pad, D), emb_table.dtype),
        grid_spec=pltpu.PrefetchScalarGridSpec(
            num_scalar_prefetch=1,
            grid=(n_pad // block_tokens,),
            in_specs=[pl.BlockSpec(memory_space=pl.ANY)],
            out_specs=pl.BlockSpec((block_tokens, D), lambda i, tok: (i, 0)),
            scratch_shapes=[pltpu.SemaphoreType.DMA],
        ),
        compiler_params=pltpu.CompilerParams(
            dimension_semantics=("parallel",),
            vmem_limit_bytes=48 << 20,
            disable_bounds_checks=True,
        ),
    )(flat_tokens, emb_table)


def kernel(tokens, emb_table):
    B, S = tokens.shape
    V, D = emb_table.shape
    N = B * S
    scale = float(D) ** 0.5

    block_tokens = 8192
    while block_tokens > N and block_tokens > 8:
        block_tokens //= 2
    chunk = min(256, block_tokens)
    unroll = 64 if block_tokens % 64 == 0 else 1

    n_pad = _round_up(N, block_tokens)
    flat = tokens.reshape(N).astype(jnp.int32)
    if n_pad != N:
        flat = jnp.concatenate([flat, jnp.zeros((n_pad - N,), jnp.int32)])

    out_flat = _embed_gather(flat, emb_table, block_tokens=block_tokens,
                             chunk=chunk, scale=scale, unroll=unroll)
    return out_flat[:N].reshape(B, S, D)


# speedup vs baseline: 1.0388x; 1.0067x over previous
import functools

import jax
import jax.numpy as jnp
from jax.experimental import pallas as pl
from jax.experimental.pallas import tpu as pltpu


def _round_up(x: int, m: int) -> int:
    return (x + m - 1) // m * m


def _gather_block_kernel(tok_ref, emb_hbm, out_ref, sem, *,
                         scale, block_tokens, chunk, unroll):
    base = pl.program_id(0) * block_tokens

    @pl.loop(0, block_tokens // unroll)
    def _(g):
        t0 = g * unroll
        toks = [tok_ref[base + t0 + u] for u in range(unroll)]
        for u in range(unroll):
            pltpu.make_async_copy(emb_hbm.at[toks[u]], out_ref.at[t0 + u],
                                  sem).start()

    pltpu.make_async_copy(emb_hbm.at[pl.ds(0, block_tokens)],
                          out_ref.at[...], sem).wait()
    out_ref[...] = out_ref[...] * scale


def _embed_gather(flat_tokens, emb_table, *, block_tokens, chunk, scale,
                  unroll):
    n_pad = flat_tokens.shape[0]
    V, D = emb_table.shape
    n_chunks = block_tokens // chunk
    return pl.pallas_call(
        functools.partial(_gather_block_kernel, scale=scale,
                          block_tokens=block_tokens, chunk=chunk,
                          unroll=unroll),
        out_shape=jax.ShapeDtypeStruct((n_pad, D), emb_table.dtype),
        grid_spec=pltpu.PrefetchScalarGridSpec(
            num_scalar_prefetch=1,
            grid=(n_pad // block_tokens,),
            in_specs=[pl.BlockSpec(memory_space=pl.ANY)],
            out_specs=pl.BlockSpec((block_tokens, D), lambda i, tok: (i, 0)),
            scratch_shapes=[pltpu.SemaphoreType.DMA],
        ),
        compiler_params=pltpu.CompilerParams(
            dimension_semantics=("parallel",),
            vmem_limit_bytes=48 << 20,
            disable_bounds_checks=True,
        ),
    )(flat_tokens, emb_table)


def kernel(tokens, emb_table):
    B, S = tokens.shape
    V, D = emb_table.shape
    N = B * S
    scale = float(D) ** 0.5

    block_tokens = 8192
    while block_tokens > N and block_tokens > 8:
        block_tokens //= 2
    chunk = min(256, block_tokens)
    unroll = 128 if block_tokens % 128 == 0 else 1

    n_pad = _round_up(N, block_tokens)
    flat = tokens.reshape(N).astype(jnp.int32)
    if n_pad != N:
        flat = jnp.concatenate([flat, jnp.zeros((n_pad - N,), jnp.int32)])

    out_flat = _embed_gather(flat, emb_table, block_tokens=block_tokens,
                             chunk=chunk, scale=scale, unroll=unroll)
    return out_flat[:N].reshape(B, S, D)


# speedup vs baseline: 1.0412x; 1.0024x over previous
import functools

import jax
import jax.numpy as jnp
from jax.experimental import pallas as pl
from jax.experimental.pallas import tpu as pltpu


def _round_up(x: int, m: int) -> int:
    return (x + m - 1) // m * m


def _gather_block_kernel(tok_ref, emb_hbm, out_ref, sem, *,
                         scale, block_tokens, chunk, unroll):
    base = pl.program_id(0) * block_tokens

    @pl.loop(0, block_tokens // unroll)
    def _(g):
        t0 = g * unroll
        toks = [tok_ref[base + t0 + u] for u in range(unroll)]
        for u in range(unroll):
            pltpu.make_async_copy(emb_hbm.at[toks[u]], out_ref.at[t0 + u],
                                  sem).start()

    pltpu.make_async_copy(emb_hbm.at[pl.ds(0, block_tokens)],
                          out_ref.at[...], sem).wait()
    out_ref[...] = out_ref[...] * scale


def _embed_gather(flat_tokens, emb_table, *, block_tokens, chunk, scale,
                  unroll):
    n_pad = flat_tokens.shape[0]
    V, D = emb_table.shape
    n_chunks = block_tokens // chunk
    return pl.pallas_call(
        functools.partial(_gather_block_kernel, scale=scale,
                          block_tokens=block_tokens, chunk=chunk,
                          unroll=unroll),
        out_shape=jax.ShapeDtypeStruct((n_pad, D), emb_table.dtype),
        grid_spec=pltpu.PrefetchScalarGridSpec(
            num_scalar_prefetch=1,
            grid=(n_pad // block_tokens,),
            in_specs=[pl.BlockSpec(memory_space=pl.ANY)],
            out_specs=pl.BlockSpec((block_tokens, D), lambda i, tok: (i, 0)),
            scratch_shapes=[pltpu.SemaphoreType.DMA],
        ),
        compiler_params=pltpu.CompilerParams(
            dimension_semantics=("parallel",),
            vmem_limit_bytes=48 << 20,
            disable_bounds_checks=True,
        ),
    )(flat_tokens, emb_table)


def kernel(tokens, emb_table):
    B, S = tokens.shape
    V, D = emb_table.shape
    N = B * S
    scale = float(D) ** 0.5

    block_tokens = 8192
    while block_tokens > N and block_tokens > 8:
        block_tokens //= 2
    chunk = min(256, block_tokens)
    unroll = 256 if block_tokens % 256 == 0 else 1

    n_pad = _round_up(N, block_tokens)
    flat = tokens.reshape(N).astype(jnp.int32)
    if n_pad != N:
        flat = jnp.concatenate([flat, jnp.zeros((n_pad - N,), jnp.int32)])

    out_flat = _embed_gather(flat, emb_table, block_tokens=block_tokens,
                             chunk=chunk, scale=scale, unroll=unroll)
    return out_flat[:N].reshape(B, S, D)
